# Initial kernel scaffold; baseline (speedup 1.0000x reference)
import jax
import jax.numpy as jnp
from jax.experimental import pallas as pl


def kernel(x, edge_index, batch, node_num, edge_num, start_node, gid, checkStatus, W1, b1, g1, be1, W2, b2, g2, be2, W3, b3, g3, be3, Wl1, bl1, Wl2, bl2):
    raise NotImplementedError("write your pallas kernel here")


# speedup vs baseline: 2.5911x; 2.5911x over previous
import functools

import jax
import jax.numpy as jnp
from jax import lax
from jax.experimental import pallas as pl
from jax.experimental.pallas import tpu as pltpu
from jax.experimental.pallas import tpu_sc as plsc

N = 10000
E = 320000
D_IN = 128
DIM = 256
NUM_GRAPHS = 16

NC, NS = 2, 16
N_PAD = 10240
E_PAD = 327680
CHUNK = 128
IB = 8
ROWS_PER_TILE = N_PAD // NS
BN = 1024
GRID = N_PAD // BN


@functools.lru_cache(maxsize=None)
def _make_sc_agg(table_rows, n_chunks):
    mesh = plsc.VectorSubcoreMesh(core_axis_name="c", subcore_axis_name="s",
                                  num_cores=NC, num_subcores=NS)

    n_slabs = n_chunks // IB

    @functools.partial(
        pl.kernel,
        out_type=jax.ShapeDtypeStruct((2, N_PAD, 128), jnp.float32),
        mesh=mesh,
        scratch_types=[
            pltpu.VMEM((IB, CHUNK), jnp.int32),
            pltpu.VMEM((IB, CHUNK), jnp.int32),
            pltpu.VMEM((CHUNK, 128), jnp.float32),
            pltpu.VMEM_SHARED((N_PAD, 128), jnp.float32),
            pltpu.SemaphoreType.DMA,
        ],
    )
    def sc_agg(table, src_idx, dst_idx, zblk, out, src_v, dst_v, rows_v, acc, sem):
        c = lax.axis_index("c")
        s = lax.axis_index("s")
        pltpu.sync_copy(zblk, acc.at[pl.ds(s * ROWS_PER_TILE, ROWS_PER_TILE)])
        plsc.subcore_barrier()

        def slab_body(o, carry):
            pltpu.sync_copy(src_idx.at[c, s, pl.ds(o * IB, IB)], src_v)
            pltpu.sync_copy(dst_idx.at[c, s, pl.ds(o * IB, IB)], dst_v)
            for j in range(IB):
                pltpu.async_copy(table.at[src_v.at[j]], rows_v, sem).wait()
                pltpu.sync_copy(rows_v, acc.at[dst_v.at[j]], add=True)
            return carry

        lax.fori_loop(0, n_slabs, slab_body, 0)
        plsc.subcore_barrier()
        pltpu.sync_copy(
            acc.at[pl.ds(s * ROWS_PER_TILE, ROWS_PER_TILE)],
            out.at[c, pl.ds(s * ROWS_PER_TILE, ROWS_PER_TILE)],
        )

    return sc_agg


def _sc_agg_l1(*args):
    return _make_sc_agg(N_PAD, E_PAD // (NC * NS) // CHUNK)(*args)


def _sc_agg_l23(*args):
    return _make_sc_agg(2 * N_PAD, E_PAD // NS // CHUNK)(*args)


def _tc_l1_body(x_ref, a_ref, w_ref, b_ref, o_ref):
    m = x_ref[...] + a_ref[0] + a_ref[1]
    y = jnp.dot(m, w_ref[...], preferred_element_type=jnp.float32) + b_ref[...]
    y = jnp.maximum(y, 0.0)
    o_ref[0] = y[:, :128]
    o_ref[1] = y[:, 128:]


def _tc_l2_body(h_ref, a_ref, wa_ref, wb_ref, b_ref, o_ref):
    m0 = h_ref[0] + a_ref[0]
    m1 = h_ref[1] + a_ref[1]
    y = (jnp.dot(m0, wa_ref[...], preferred_element_type=jnp.float32)
         + jnp.dot(m1, wb_ref[...], preferred_element_type=jnp.float32)
         + b_ref[...])
    y = jnp.maximum(y, 0.0)
    o_ref[0] = y[:, :128]
    o_ref[1] = y[:, 128:]


def _tc_l3_body(h_ref, a_ref, wa_ref, wb_ref, b_ref, wl1_ref, bl1_ref,
                wl2_ref, bl2_ref, batch_ref, o_ref, sums_acc, cnts_acc):
    i = pl.program_id(0)
    m0 = h_ref[0] + a_ref[0]
    m1 = h_ref[1] + a_ref[1]
    y = (jnp.dot(m0, wa_ref[...], preferred_element_type=jnp.float32)
         + jnp.dot(m1, wb_ref[...], preferred_element_type=jnp.float32)
         + b_ref[...])
    t = jnp.dot(y, wl1_ref[...], preferred_element_type=jnp.float32) + bl1_ref[...]
    t = jnp.maximum(t, 0.0)
    sraw = jnp.dot(t, wl2_ref[...], preferred_element_type=jnp.float32) + bl2_ref[...]
    sig = jax.nn.sigmoid(sraw)
    gids = lax.broadcasted_iota(jnp.int32, (BN, NUM_GRAPHS), 1)
    oh = (batch_ref[...] == gids).astype(jnp.float32)
    ssum = jnp.sum(oh * sig, axis=0, keepdims=True)
    scnt = jnp.sum(oh, axis=0, keepdims=True)

    @pl.when(i == 0)
    def _():
        sums_acc[...] = ssum
        cnts_acc[...] = scnt

    @pl.when(i > 0)
    def _():
        sums_acc[...] += ssum
        cnts_acc[...] += scnt

    @pl.when(i == pl.num_programs(0) - 1)
    def _():
        o_ref[...] = sums_acc[...] / jnp.maximum(cnts_acc[...], 1.0)


def _rep(shape):
    return pl.BlockSpec(shape, lambda i: tuple(0 for _ in shape))


_tc_l1 = pl.pallas_call(
    _tc_l1_body,
    grid=(GRID,),
    in_specs=[
        pl.BlockSpec((BN, 128), lambda i: (i, 0)),
        pl.BlockSpec((2, BN, 128), lambda i: (0, i, 0)),
        _rep((128, 256)),
        _rep((1, 256)),
    ],
    out_specs=pl.BlockSpec((2, BN, 128), lambda i: (0, i, 0)),
    out_shape=jax.ShapeDtypeStruct((2, N_PAD, 128), jnp.float32),
)

_tc_l2 = pl.pallas_call(
    _tc_l2_body,
    grid=(GRID,),
    in_specs=[
        pl.BlockSpec((2, BN, 128), lambda i: (0, i, 0)),
        pl.BlockSpec((2, BN, 128), lambda i: (0, i, 0)),
        _rep((128, 256)),
        _rep((128, 256)),
        _rep((1, 256)),
    ],
    out_specs=pl.BlockSpec((2, BN, 128), lambda i: (0, i, 0)),
    out_shape=jax.ShapeDtypeStruct((2, N_PAD, 128), jnp.float32),
)

_tc_l3 = pl.pallas_call(
    _tc_l3_body,
    grid=(GRID,),
    in_specs=[
        pl.BlockSpec((2, BN, 128), lambda i: (0, i, 0)),
        pl.BlockSpec((2, BN, 128), lambda i: (0, i, 0)),
        _rep((128, 256)),
        _rep((128, 256)),
        _rep((1, 256)),
        _rep((256, 128)),
        _rep((1, 128)),
        _rep((128, 1)),
        _rep((1, 1)),
        pl.BlockSpec((BN, 1), lambda i: (i, 0)),
    ],
    out_specs=_rep((1, NUM_GRAPHS)),
    out_shape=jax.ShapeDtypeStruct((1, NUM_GRAPHS), jnp.float32),
    scratch_shapes=[
        pltpu.VMEM((1, NUM_GRAPHS), jnp.float32),
        pltpu.VMEM((1, NUM_GRAPHS), jnp.float32),
    ],
)


def kernel(x, edge_index, batch, node_num, edge_num, start_node, gid,
           checkStatus, W1, b1, g1, be1, W2, b2, g2, be2, W3, b3, g3, be3,
           Wl1, bl1, Wl2, bl2):
    del node_num, edge_num, start_node, gid, checkStatus
    f32 = jnp.float32
    bn_scale = 1.0 / jnp.sqrt(jnp.float32(1.0 + 1e-5))

    def fold(W, b, g, be):
        sc = g * bn_scale
        return W * sc[None, :], (b * sc + be)[None, :]

    W1f, b1f = fold(W1, b1, g1, be1)
    W2f, b2f = fold(W2, b2, g2, be2)
    W3f, b3f = fold(W3, b3, g3, be3)

    x_pad = jnp.pad(x, ((0, N_PAD - N), (0, 0)))
    src = jnp.pad(edge_index[0], (0, E_PAD - E), constant_values=N)
    dst = jnp.pad(edge_index[1], (0, E_PAD - E), constant_values=N)

    src1 = src.reshape(2, NS, -1, CHUNK)
    dst1 = dst.reshape(2, NS, -1, CHUNK)
    src_r = src.reshape(NS, -1, CHUNK)
    dst_r = dst.reshape(NS, -1, CHUNK)
    src23 = jnp.stack([src_r, src_r + N_PAD])
    dst23 = jnp.stack([dst_r, dst_r])
    zblk = jnp.zeros((ROWS_PER_TILE, 128), f32)

    agg1 = _sc_agg_l1(x_pad, src1, dst1, zblk)
    h1 = _tc_l1(x_pad, agg1, W1f, b1f)
    agg2 = _sc_agg_l23(h1.reshape(2 * N_PAD, 128), src23, dst23, zblk)
    h2 = _tc_l2(h1, agg2, W2f[:128], W2f[128:], b2f)
    agg3 = _sc_agg_l23(h2.reshape(2 * N_PAD, 128), src23, dst23, zblk)

    batch_pad = jnp.pad(batch, (0, N_PAD - N), constant_values=-1)
    pooled = _tc_l3(h2, agg3, W3f[:128], W3f[128:], b3f,
                    Wl1, bl1[None, :], Wl2, bl2[None, :],
                    batch_pad.reshape(N_PAD, 1))
    return pooled.reshape(NUM_GRAPHS, 1)


# speedup vs baseline: 3.1200x; 1.2041x over previous
import functools

import jax
import jax.numpy as jnp
from jax import lax
from jax.experimental import pallas as pl
from jax.experimental.pallas import tpu as pltpu
from jax.experimental.pallas import tpu_sc as plsc

N = 10000
E = 320000
D_IN = 128
DIM = 256
NUM_GRAPHS = 16

NC, NS = 2, 16
N_PAD = 10240
E_PAD = 327680
CHUNK = 128
IB = 8
ROWS_PER_TILE = N_PAD // NS
BN = 1024
GRID = N_PAD // BN


@functools.lru_cache(maxsize=None)
def _make_sc_agg(table_rows, n_chunks):
    mesh = plsc.VectorSubcoreMesh(core_axis_name="c", subcore_axis_name="s",
                                  num_cores=NC, num_subcores=NS)

    n_slabs = n_chunks // IB
    assert n_slabs % 2 == 0 and n_slabs >= 4 and IB % 2 == 0

    @functools.partial(
        pl.kernel,
        out_type=jax.ShapeDtypeStruct((2, N_PAD, 128), jnp.float32),
        mesh=mesh,
        scratch_types=[
            pltpu.VMEM((2 * IB, CHUNK), jnp.int32),
            pltpu.VMEM((2 * IB, CHUNK), jnp.int32),
            pltpu.VMEM((CHUNK, 128), jnp.float32),
            pltpu.VMEM((CHUNK, 128), jnp.float32),
            pltpu.VMEM_SHARED((N_PAD, 128), jnp.float32),
            pltpu.SemaphoreType.DMA,
            pltpu.SemaphoreType.DMA,
            pltpu.SemaphoreType.DMA,
            pltpu.SemaphoreType.DMA,
            pltpu.SemaphoreType.DMA,
            pltpu.SemaphoreType.DMA,
        ],
    )
    def sc_agg(table, idx_hbm, zblk, out, ib0, ib1, r0, r1, acc,
               gs0, gs1, ss0, ss1, is0, is1):
        idxb, rows = (ib0, ib1), (r0, r1)
        gsem, ssem, isem = (gs0, gs1), (ss0, ss1), (is0, is1)
        c = lax.axis_index("c")
        s = lax.axis_index("s")

        def wait_rows(sem):
            pltpu.make_async_copy(table.at[pl.ds(0, CHUNK)], rows[0], sem).wait()

        def wait_slab(sem):
            pltpu.make_async_copy(idx_hbm.at[c, s, 0], idxb[0], sem).wait()

        pltpu.sync_copy(zblk, acc.at[pl.ds(s * ROWS_PER_TILE, ROWS_PER_TILE)])
        pltpu.sync_copy(idx_hbm.at[c, s, 0], idxb[0])
        plsc.subcore_barrier()
        pltpu.async_copy(table.at[idxb[0].at[0]], rows[0], gsem[0])

        def do_slab(o, p, first, last):
            if not last:
                pltpu.async_copy(idx_hbm.at[c, s, o + 1], idxb[1 - p],
                                 isem[1 - p])
            for jl in range(IB):
                b = jl % 2
                if not (first and jl == 0):
                    wait_rows(ssem[1 - b])
                if not (last and jl == IB - 1):
                    if jl == IB - 1:
                        wait_slab(isem[1 - p])
                        nb, njl = 1 - p, 0
                    else:
                        nb, njl = p, jl + 1
                    pltpu.async_copy(table.at[idxb[nb].at[njl]], rows[1 - b],
                                     gsem[1 - b])
                wait_rows(gsem[b])
                pltpu.async_copy(rows[b], acc.at[idxb[p].at[IB + jl]], ssem[b],
                                 add=True)

        do_slab(0, 0, True, False)
        do_slab(1, 1, False, False)

        def pair_body(k, carry):
            o = 2 * k
            do_slab(o, 0, False, False)
            do_slab(o + 1, 1, False, False)
            return carry

        lax.fori_loop(1, n_slabs // 2 - 1, pair_body, 0)
        do_slab(n_slabs - 2, 0, False, False)
        do_slab(n_slabs - 1, 1, False, True)
        wait_rows(ssem[1])

        plsc.subcore_barrier()
        pltpu.sync_copy(
            acc.at[pl.ds(s * ROWS_PER_TILE, ROWS_PER_TILE)],
            out.at[c, pl.ds(s * ROWS_PER_TILE, ROWS_PER_TILE)],
        )

    return sc_agg


def _sc_agg_l1(*args):
    return _make_sc_agg(N_PAD, E_PAD // (NC * NS) // CHUNK)(*args)


def _sc_agg_l23(*args):
    return _make_sc_agg(2 * N_PAD, E_PAD // NS // CHUNK)(*args)


def _tc_l1_body(x_ref, a_ref, w_ref, b_ref, o_ref):
    m = x_ref[...] + a_ref[0] + a_ref[1]
    y = jnp.dot(m, w_ref[...], preferred_element_type=jnp.float32) + b_ref[...]
    y = jnp.maximum(y, 0.0)
    o_ref[0] = y[:, :128]
    o_ref[1] = y[:, 128:]


def _tc_l2_body(h_ref, a_ref, wa_ref, wb_ref, b_ref, o_ref):
    m0 = h_ref[0] + a_ref[0]
    m1 = h_ref[1] + a_ref[1]
    y = (jnp.dot(m0, wa_ref[...], preferred_element_type=jnp.float32)
         + jnp.dot(m1, wb_ref[...], preferred_element_type=jnp.float32)
         + b_ref[...])
    y = jnp.maximum(y, 0.0)
    o_ref[0] = y[:, :128]
    o_ref[1] = y[:, 128:]


def _tc_l3_body(h_ref, a_ref, wa_ref, wb_ref, b_ref, wl1_ref, bl1_ref,
                wl2_ref, bl2_ref, batch_ref, o_ref, sums_acc, cnts_acc):
    i = pl.program_id(0)
    m0 = h_ref[0] + a_ref[0]
    m1 = h_ref[1] + a_ref[1]
    y = (jnp.dot(m0, wa_ref[...], preferred_element_type=jnp.float32)
         + jnp.dot(m1, wb_ref[...], preferred_element_type=jnp.float32)
         + b_ref[...])
    t = jnp.dot(y, wl1_ref[...], preferred_element_type=jnp.float32) + bl1_ref[...]
    t = jnp.maximum(t, 0.0)
    sraw = jnp.dot(t, wl2_ref[...], preferred_element_type=jnp.float32) + bl2_ref[...]
    sig = jax.nn.sigmoid(sraw)
    gids = lax.broadcasted_iota(jnp.int32, (BN, NUM_GRAPHS), 1)
    oh = (batch_ref[...] == gids).astype(jnp.float32)
    ssum = jnp.sum(oh * sig, axis=0, keepdims=True)
    scnt = jnp.sum(oh, axis=0, keepdims=True)

    @pl.when(i == 0)
    def _():
        sums_acc[...] = ssum
        cnts_acc[...] = scnt

    @pl.when(i > 0)
    def _():
        sums_acc[...] += ssum
        cnts_acc[...] += scnt

    @pl.when(i == pl.num_programs(0) - 1)
    def _():
        o_ref[...] = sums_acc[...] / jnp.maximum(cnts_acc[...], 1.0)


def _rep(shape):
    return pl.BlockSpec(shape, lambda i: tuple(0 for _ in shape))


_tc_l1 = pl.pallas_call(
    _tc_l1_body,
    grid=(GRID,),
    in_specs=[
        pl.BlockSpec((BN, 128), lambda i: (i, 0)),
        pl.BlockSpec((2, BN, 128), lambda i: (0, i, 0)),
        _rep((128, 256)),
        _rep((1, 256)),
    ],
    out_specs=pl.BlockSpec((2, BN, 128), lambda i: (0, i, 0)),
    out_shape=jax.ShapeDtypeStruct((2, N_PAD, 128), jnp.float32),
)

_tc_l2 = pl.pallas_call(
    _tc_l2_body,
    grid=(GRID,),
    in_specs=[
        pl.BlockSpec((2, BN, 128), lambda i: (0, i, 0)),
        pl.BlockSpec((2, BN, 128), lambda i: (0, i, 0)),
        _rep((128, 256)),
        _rep((128, 256)),
        _rep((1, 256)),
    ],
    out_specs=pl.BlockSpec((2, BN, 128), lambda i: (0, i, 0)),
    out_shape=jax.ShapeDtypeStruct((2, N_PAD, 128), jnp.float32),
)

_tc_l3 = pl.pallas_call(
    _tc_l3_body,
    grid=(GRID,),
    in_specs=[
        pl.BlockSpec((2, BN, 128), lambda i: (0, i, 0)),
        pl.BlockSpec((2, BN, 128), lambda i: (0, i, 0)),
        _rep((128, 256)),
        _rep((128, 256)),
        _rep((1, 256)),
        _rep((256, 128)),
        _rep((1, 128)),
        _rep((128, 1)),
        _rep((1, 1)),
        pl.BlockSpec((BN, 1), lambda i: (i, 0)),
    ],
    out_specs=_rep((1, NUM_GRAPHS)),
    out_shape=jax.ShapeDtypeStruct((1, NUM_GRAPHS), jnp.float32),
    scratch_shapes=[
        pltpu.VMEM((1, NUM_GRAPHS), jnp.float32),
        pltpu.VMEM((1, NUM_GRAPHS), jnp.float32),
    ],
)


def kernel(x, edge_index, batch, node_num, edge_num, start_node, gid,
           checkStatus, W1, b1, g1, be1, W2, b2, g2, be2, W3, b3, g3, be3,
           Wl1, bl1, Wl2, bl2):
    del node_num, edge_num, start_node, gid, checkStatus
    f32 = jnp.float32
    bn_scale = 1.0 / jnp.sqrt(jnp.float32(1.0 + 1e-5))

    def fold(W, b, g, be):
        sc = g * bn_scale
        return W * sc[None, :], (b * sc + be)[None, :]

    W1f, b1f = fold(W1, b1, g1, be1)
    W2f, b2f = fold(W2, b2, g2, be2)
    W3f, b3f = fold(W3, b3, g3, be3)

    x_pad = jnp.pad(x, ((0, N_PAD - N), (0, 0)))
    src = jnp.pad(edge_index[0], (0, E_PAD - E), constant_values=N)
    dst = jnp.pad(edge_index[1], (0, E_PAD - E), constant_values=N)

    def slabify(s_arr, d_arr):
        s4 = s_arr.reshape(2, NS, -1, IB, CHUNK)
        d4 = d_arr.reshape(2, NS, -1, IB, CHUNK)
        return jnp.concatenate([s4, d4], axis=3)

    idx1 = slabify(src.reshape(2, NS, -1, CHUNK), dst.reshape(2, NS, -1, CHUNK))
    src_r = src.reshape(NS, -1, CHUNK)
    dst_r = dst.reshape(NS, -1, CHUNK)
    idx23 = slabify(jnp.stack([src_r, src_r + N_PAD]),
                    jnp.stack([dst_r, dst_r]))
    zblk = jnp.zeros((ROWS_PER_TILE, 128), f32)

    agg1 = _sc_agg_l1(x_pad, idx1, zblk)
    h1 = _tc_l1(x_pad, agg1, W1f, b1f)
    agg2 = _sc_agg_l23(h1.reshape(2 * N_PAD, 128), idx23, zblk)
    h2 = _tc_l2(h1, agg2, W2f[:128], W2f[128:], b2f)
    agg3 = _sc_agg_l23(h2.reshape(2 * N_PAD, 128), idx23, zblk)

    batch_pad = jnp.pad(batch, (0, N_PAD - N), constant_values=-1)
    pooled = _tc_l3(h2, agg3, W3f[:128], W3f[128:], b3f,
                    Wl1, bl1[None, :], Wl2, bl2[None, :],
                    batch_pad.reshape(N_PAD, 1))
    return pooled.reshape(NUM_GRAPHS, 1)


# speedup vs baseline: 9.4582x; 3.0315x over previous
import functools

import jax
import jax.numpy as jnp
from jax import lax
from jax.experimental import pallas as pl
from jax.experimental.pallas import tpu as pltpu
from jax.experimental.pallas import tpu_sc as plsc

N = 10000
E = 320000
D_IN = 128
DIM = 256
NUM_GRAPHS = 16

NC, NS = 2, 16
N_PAD = 10240
E_PAD = 327680
CHUNK = 128
IB = 8
ROWS_PER_TILE = N_PAD // NS
BN = 1024
GRID = N_PAD // BN


@functools.lru_cache(maxsize=None)
def _make_sc_agg(table_rows, n_chunks):
    mesh = plsc.VectorSubcoreMesh(core_axis_name="c", subcore_axis_name="s",
                                  num_cores=NC, num_subcores=NS)

    n_slabs = n_chunks // IB
    assert n_slabs % 2 == 0 and n_slabs >= 4 and IB % 2 == 0

    @functools.partial(
        pl.kernel,
        out_type=jax.ShapeDtypeStruct((2, N_PAD, 128), jnp.float32),
        mesh=mesh,
        scratch_types=[
            pltpu.VMEM((2 * IB, CHUNK), jnp.int32),
            pltpu.VMEM((2 * IB, CHUNK), jnp.int32),
            pltpu.VMEM((CHUNK, 128), jnp.float32),
            pltpu.VMEM((CHUNK, 128), jnp.float32),
            pltpu.VMEM_SHARED((N_PAD, 128), jnp.float32),
            pltpu.SemaphoreType.DMA,
            pltpu.SemaphoreType.DMA,
            pltpu.SemaphoreType.DMA,
            pltpu.SemaphoreType.DMA,
            pltpu.SemaphoreType.DMA,
            pltpu.SemaphoreType.DMA,
        ],
    )
    def sc_agg(table, idx_hbm, zblk, out, ib0, ib1, r0, r1, acc,
               gs0, gs1, ss0, ss1, is0, is1):
        idxb, rows = (ib0, ib1), (r0, r1)
        gsem, ssem, isem = (gs0, gs1), (ss0, ss1), (is0, is1)
        c = lax.axis_index("c")
        s = lax.axis_index("s")

        def wait_rows(sem):
            pltpu.make_async_copy(table.at[pl.ds(0, CHUNK)], rows[0], sem).wait()

        def wait_slab(sem):
            pltpu.make_async_copy(idx_hbm.at[c, s, 0], idxb[0], sem).wait()

        pltpu.sync_copy(zblk, acc.at[pl.ds(s * ROWS_PER_TILE, ROWS_PER_TILE)])
        pltpu.sync_copy(idx_hbm.at[c, s, 0], idxb[0])
        plsc.subcore_barrier()
        pltpu.async_copy(table.at[idxb[0].at[0]], rows[0], gsem[0])

        def do_slab(o, p, first, last):
            if not last:
                pltpu.async_copy(idx_hbm.at[c, s, o + 1], idxb[1 - p],
                                 isem[1 - p])
            for jl in range(IB):
                b = jl % 2
                if not (first and jl == 0):
                    wait_rows(ssem[1 - b])
                if not (last and jl == IB - 1):
                    if jl == IB - 1:
                        wait_slab(isem[1 - p])
                        nb, njl = 1 - p, 0
                    else:
                        nb, njl = p, jl + 1
                    pltpu.async_copy(table.at[idxb[nb].at[njl]], rows[1 - b],
                                     gsem[1 - b])
                wait_rows(gsem[b])
                pltpu.async_copy(rows[b], acc.at[idxb[p].at[IB + jl]], ssem[b],
                                 add=True)

        do_slab(0, 0, True, False)
        do_slab(1, 1, False, False)

        def pair_body(k, carry):
            o = 2 * k
            do_slab(o, 0, False, False)
            do_slab(o + 1, 1, False, False)
            return carry

        lax.fori_loop(1, n_slabs // 2 - 1, pair_body, 0)
        do_slab(n_slabs - 2, 0, False, False)
        do_slab(n_slabs - 1, 1, False, True)
        wait_rows(ssem[1])

        plsc.subcore_barrier()
        pltpu.sync_copy(
            acc.at[pl.ds(s * ROWS_PER_TILE, ROWS_PER_TILE)],
            out.at[c, pl.ds(s * ROWS_PER_TILE, ROWS_PER_TILE)],
        )

    return sc_agg


def _sc_agg_l1(*args):
    return _make_sc_agg(N_PAD, E_PAD // (NC * NS) // CHUNK)(*args)


def _sc_agg_l23(*args):
    return _make_sc_agg(2 * N_PAD, E_PAD // NS // CHUNK)(*args)


def _tc_l1_body(x_ref, a_ref, w_ref, b_ref, o_ref):
    m = x_ref[...] + a_ref[0] + a_ref[1]
    y = jnp.dot(m, w_ref[...], preferred_element_type=jnp.float32) + b_ref[...]
    y = jnp.maximum(y, 0.0)
    o_ref[0] = y[:, :128]
    o_ref[1] = y[:, 128:]


def _tc_l2_body(h_ref, a_ref, wa_ref, wb_ref, b_ref, o_ref):
    m0 = h_ref[0] + a_ref[0]
    m1 = h_ref[1] + a_ref[1]
    y = (jnp.dot(m0, wa_ref[...], preferred_element_type=jnp.float32)
         + jnp.dot(m1, wb_ref[...], preferred_element_type=jnp.float32)
         + b_ref[...])
    y = jnp.maximum(y, 0.0)
    o_ref[0] = y[:, :128]
    o_ref[1] = y[:, 128:]


def _tc_l3_body(h_ref, a_ref, wa_ref, wb_ref, b_ref, wl1_ref, bl1_ref,
                wl2_ref, bl2_ref, batch_ref, o_ref, sums_acc, cnts_acc):
    i = pl.program_id(0)
    m0 = h_ref[0] + a_ref[0]
    m1 = h_ref[1] + a_ref[1]
    y = (jnp.dot(m0, wa_ref[...], preferred_element_type=jnp.float32)
         + jnp.dot(m1, wb_ref[...], preferred_element_type=jnp.float32)
         + b_ref[...])
    t = jnp.dot(y, wl1_ref[...], preferred_element_type=jnp.float32) + bl1_ref[...]
    t = jnp.maximum(t, 0.0)
    sraw = jnp.dot(t, wl2_ref[...], preferred_element_type=jnp.float32) + bl2_ref[...]
    sig = jax.nn.sigmoid(sraw)
    gids = lax.broadcasted_iota(jnp.int32, (BN, NUM_GRAPHS), 1)
    oh = (batch_ref[...] == gids).astype(jnp.float32)
    ssum = jnp.sum(oh * sig, axis=0, keepdims=True)
    scnt = jnp.sum(oh, axis=0, keepdims=True)

    @pl.when(i == 0)
    def _():
        sums_acc[...] = ssum
        cnts_acc[...] = scnt

    @pl.when(i > 0)
    def _():
        sums_acc[...] += ssum
        cnts_acc[...] += scnt

    @pl.when(i == pl.num_programs(0) - 1)
    def _():
        o_ref[...] = sums_acc[...] / jnp.maximum(cnts_acc[...], 1.0)


def _rep(shape):
    return pl.BlockSpec(shape, lambda i: tuple(0 for _ in shape))


_tc_l1 = pl.pallas_call(
    _tc_l1_body,
    grid=(GRID,),
    in_specs=[
        pl.BlockSpec((BN, 128), lambda i: (i, 0)),
        pl.BlockSpec((2, BN, 128), lambda i: (0, i, 0)),
        _rep((128, 256)),
        _rep((1, 256)),
    ],
    out_specs=pl.BlockSpec((2, BN, 128), lambda i: (0, i, 0)),
    out_shape=jax.ShapeDtypeStruct((2, N_PAD, 128), jnp.float32),
)

_tc_l2 = pl.pallas_call(
    _tc_l2_body,
    grid=(GRID,),
    in_specs=[
        pl.BlockSpec((2, BN, 128), lambda i: (0, i, 0)),
        pl.BlockSpec((2, BN, 128), lambda i: (0, i, 0)),
        _rep((128, 256)),
        _rep((128, 256)),
        _rep((1, 256)),
    ],
    out_specs=pl.BlockSpec((2, BN, 128), lambda i: (0, i, 0)),
    out_shape=jax.ShapeDtypeStruct((2, N_PAD, 128), jnp.float32),
)

_tc_l3 = pl.pallas_call(
    _tc_l3_body,
    grid=(GRID,),
    in_specs=[
        pl.BlockSpec((2, BN, 128), lambda i: (0, i, 0)),
        pl.BlockSpec((2, BN, 128), lambda i: (0, i, 0)),
        _rep((128, 256)),
        _rep((128, 256)),
        _rep((1, 256)),
        _rep((256, 128)),
        _rep((1, 128)),
        _rep((128, 1)),
        _rep((1, 1)),
        pl.BlockSpec((BN, 1), lambda i: (i, 0)),
    ],
    out_specs=_rep((1, NUM_GRAPHS)),
    out_shape=jax.ShapeDtypeStruct((1, NUM_GRAPHS), jnp.float32),
    scratch_shapes=[
        pltpu.VMEM((1, NUM_GRAPHS), jnp.float32),
        pltpu.VMEM((1, NUM_GRAPHS), jnp.float32),
    ],
)


def kernel(x, edge_index, batch, node_num, edge_num, start_node, gid,
           checkStatus, W1, b1, g1, be1, W2, b2, g2, be2, W3, b3, g3, be3,
           Wl1, bl1, Wl2, bl2):
    del node_num, edge_num, start_node, gid, checkStatus
    f32 = jnp.float32
    bn_scale = 1.0 / jnp.sqrt(jnp.float32(1.0 + 1e-5))

    def fold(W, b, g, be):
        sc = g * bn_scale
        return W * sc[None, :], (b * sc + be)[None, :]

    W1f, b1f = fold(W1, b1, g1, be1)
    W2f, b2f = fold(W2, b2, g2, be2)
    W3f, b3f = fold(W3, b3, g3, be3)

    x_pad = jnp.pad(x, ((0, N_PAD - N), (0, 0)))
    pad_rows = N + jnp.arange(E_PAD - E, dtype=jnp.int32) % (N_PAD - N)
    src = jnp.concatenate([edge_index[0], pad_rows])
    dst = jnp.concatenate([edge_index[1], pad_rows])

    def slabify(s_arr, d_arr):
        s4 = s_arr.reshape(2, NS, -1, IB, CHUNK)
        d4 = d_arr.reshape(2, NS, -1, IB, CHUNK)
        return jnp.concatenate([s4, d4], axis=3)

    idx1 = slabify(src.reshape(2, NS, -1, CHUNK), dst.reshape(2, NS, -1, CHUNK))
    src_r = src.reshape(NS, -1, CHUNK)
    dst_r = dst.reshape(NS, -1, CHUNK)
    idx23 = slabify(jnp.stack([src_r, src_r + N_PAD]),
                    jnp.stack([dst_r, dst_r]))
    zblk = jnp.zeros((ROWS_PER_TILE, 128), f32)

    agg1 = _sc_agg_l1(x_pad, idx1, zblk)
    h1 = _tc_l1(x_pad, agg1, W1f, b1f)
    agg2 = _sc_agg_l23(h1.reshape(2 * N_PAD, 128), idx23, zblk)
    h2 = _tc_l2(h1, agg2, W2f[:128], W2f[128:], b2f)
    agg3 = _sc_agg_l23(h2.reshape(2 * N_PAD, 128), idx23, zblk)

    batch_pad = jnp.pad(batch, (0, N_PAD - N), constant_values=-1)
    pooled = _tc_l3(h2, agg3, W3f[:128], W3f[128:], b3f,
                    Wl1, bl1[None, :], Wl2, bl2[None, :],
                    batch_pad.reshape(N_PAD, 1))
    return pooled.reshape(NUM_GRAPHS, 1)
